# baseline (speedup 1.0000x reference)
import functools

import numpy as np
import jax
import jax.numpy as jnp
from jax import lax
from jax.experimental import pallas as pl
from jax.experimental.pallas import tpu as pltpu
from jax.experimental.pallas import tpu_sc as plsc

NUM_FIELDS = 26
VOCAB = 100000
DIM = 32
BATCH = 4096
HIST = 50

LANES = 16
HIST_PAD = 56
NROWS = NUM_FIELDS * HIST_PAD
NW = 32
B_PER_W = BATCH // NW
HALF = DIM // 2
OUT_D = NUM_FIELDS * DIM

_MESH = dict(core_axis_name="c", subcore_axis_name="s")
_PARAMS = dict(needs_layout_passes=False, use_tc_tiling_on_sc=False)


def _ids_body(x_hbm, gid_hbm, rcp_hbm, xv, gid2, rcp, lock0, lock1):
    del lock0, lock1
    wid = lax.axis_index("s") * 2 + lax.axis_index("c")
    base = wid * B_PER_W
    tail2 = lax.iota(jnp.int32, LANES) >= (LANES - 2)

    for f in range(NUM_FIELDS):
        gid2[f, pl.ds(HIST_PAD - LANES, LANES)] = jnp.zeros(
            (LANES,), jnp.int32)

    @pl.loop(0, B_PER_W)
    def _batch(b):
        bb = base + b
        pltpu.sync_copy(x_hbm.at[bb], xv)
        for f in range(NUM_FIELDS):
            xa = xv[f, pl.ds(0, LANES)]
            xb = xv[f, pl.ds(LANES, LANES)]
            xc = xv[f, pl.ds(2 * LANES, LANES)]
            xd = xv[f, pl.ds(HIST - LANES, LANES)]
            gid2[f, pl.ds(0, LANES)] = xa
            gid2[f, pl.ds(LANES, LANES)] = xb
            gid2[f, pl.ds(2 * LANES, LANES)] = xc
            gid2[f, pl.ds(HIST - LANES, LANES)] = xd
            cnt = (plsc.all_reduce_population_count(xa != 0)
                   + plsc.all_reduce_population_count(xb != 0)
                   + plsc.all_reduce_population_count(xc != 0)
                   + plsc.all_reduce_population_count((xd != 0) & tail2))
            rcp[f, pl.ds(0, LANES)] = 1.0 / jnp.maximum(
                cnt.astype(jnp.float32), 1.0)
        pltpu.sync_copy(gid2, gid_hbm.at[bb])
        pltpu.sync_copy(rcp, rcp_hbm.at[bb])


def _pool_body(gid_hbm, rcp_hbm, tab_hbm, out_hbm,
               gid0, gid1, rows0, rows1, rcp0, rcp1, outv, sem0, sem1):
    wid = lax.axis_index("s") * 2 + lax.axis_index("c")
    base = wid * B_PER_W
    zero16 = jnp.zeros((LANES,), jnp.float32)
    lane = lax.iota(jnp.int32, LANES)

    def stage(b, gid2, rcp, sem):
        bb = base + b
        pltpu.sync_copy(gid_hbm.at[bb], gid2)
        pltpu.sync_copy(rcp_hbm.at[bb], rcp)

    def fire(gid2, rows, sem):
        for f in range(NUM_FIELDS):
            pltpu.async_copy(
                tab_hbm.at[f].at[gid2.at[f]],
                rows.at[pl.ds(f * HIST_PAD, HIST_PAD)], sem)

    def drain_reduce(b, rows, rcp, sem):
        pltpu.make_async_copy(
            tab_hbm.at[0].at[pl.ds(0, NROWS)], rows, sem).wait()
        for f in range(NUM_FIELDS):
            rf = rcp[f, pl.ds(0, LANES)]

            @pl.loop(0, HIST, init_carry=(zero16, zero16), unroll=10)
            def _sum(l, carry):
                ae, ao = carry
                e, o = plsc.unpack(rows[f * HIST_PAD + l, :],
                                   format=plsc.PackFormat.INTERLEAVED)
                return ae + e, ao + o

            ae, ao = _sum
            idx_e = jnp.full((LANES,), f * DIM, jnp.int32) + 2 * lane
            plsc.store_scatter(outv, [idx_e], ae * rf)
            plsc.store_scatter(outv, [idx_e + 1], ao * rf)
        pltpu.sync_copy(outv, out_hbm.at[base + b])

    def stage_full(b, gid2, rows, rcp, sem):
        stage(b, gid2, rcp, sem)
        fire(gid2, rows, sem)

    stage_full(0, gid0, rows0, rcp0, sem0)

    @pl.loop(0, B_PER_W // 2 - 1)
    def _pair(t):
        b = 2 * t
        stage_full(b + 1, gid1, rows1, rcp1, sem1)
        drain_reduce(b, rows0, rcp0, sem0)
        stage_full(b + 2, gid0, rows0, rcp0, sem0)
        drain_reduce(b + 1, rows1, rcp1, sem1)

    stage_full(B_PER_W - 1, gid1, rows1, rcp1, sem1)
    drain_reduce(B_PER_W - 2, rows0, rcp0, sem0)
    drain_reduce(B_PER_W - 1, rows1, rcp1, sem1)


@jax.jit
def kernel(x, tables):
    ids_call = pl.kernel(
        _ids_body,
        out_type=(
            jax.ShapeDtypeStruct((BATCH, NUM_FIELDS, HIST_PAD), jnp.int32),
            jax.ShapeDtypeStruct((BATCH, NUM_FIELDS, LANES), jnp.float32),
        ),
        mesh=plsc.VectorSubcoreMesh(**_MESH),
        compiler_params=pltpu.CompilerParams(**_PARAMS),
        scratch_types=[
            pltpu.VMEM((NUM_FIELDS, HIST), jnp.int32),
            pltpu.VMEM((NUM_FIELDS, HIST_PAD), jnp.int32),
            pltpu.VMEM((NUM_FIELDS, LANES), jnp.float32),
            pltpu.SemaphoreType.DMA,
            pltpu.SemaphoreType.DMA,
        ],
    )
    gid_all, rcp_all = ids_call(x)

    pool_call = pl.kernel(
        _pool_body,
        out_type=jax.ShapeDtypeStruct((BATCH, OUT_D), jnp.float32),
        mesh=plsc.VectorSubcoreMesh(**_MESH),
        compiler_params=pltpu.CompilerParams(**_PARAMS),
        scratch_types=[
            pltpu.VMEM((NUM_FIELDS, HIST_PAD), jnp.int32),
            pltpu.VMEM((NUM_FIELDS, HIST_PAD), jnp.int32),
            pltpu.VMEM((NROWS, DIM), jnp.bfloat16),
            pltpu.VMEM((NROWS, DIM), jnp.bfloat16),
            pltpu.VMEM((NUM_FIELDS, LANES), jnp.float32),
            pltpu.VMEM((NUM_FIELDS, LANES), jnp.float32),
            pltpu.VMEM((OUT_D,), jnp.float32),
            pltpu.SemaphoreType.DMA,
            pltpu.SemaphoreType.DMA,
        ],
    )
    return pool_call(gid_all, rcp_all, tables.astype(jnp.bfloat16))


# speedup vs baseline: 1.0778x; 1.0778x over previous
import functools

import numpy as np
import jax
import jax.numpy as jnp
from jax import lax
from jax.experimental import pallas as pl
from jax.experimental.pallas import tpu as pltpu
from jax.experimental.pallas import tpu_sc as plsc

NUM_FIELDS = 26
VOCAB = 100000
DIM = 32
BATCH = 4096
HIST = 50

LANES = 16
HIST_PAD = 56
NW = 32
B_PER_W = BATCH // NW
HALF = DIM // 2
OUT_D = NUM_FIELDS * DIM


def _body(x_hbm, tab_hbm, out_hbm,
          xv0, xv1, gid0, gid1, rows0, rows1, rcp0, rcp1, outv,
          sem0, sem1):
    wid = lax.axis_index("s") * 2 + lax.axis_index("c")
    base = wid * B_PER_W
    zero16 = jnp.zeros((LANES,), jnp.float32)
    tail2 = lax.iota(jnp.int32, LANES) >= (LANES - 2)

    def stage(b, xv, gid2, rcp, sem):
        pltpu.sync_copy(x_hbm.at[base + b], xv)
        for f in range(NUM_FIELDS):
            xa = xv[f, pl.ds(0, LANES)]
            xb = xv[f, pl.ds(LANES, LANES)]
            xc = xv[f, pl.ds(2 * LANES, LANES)]
            xd = xv[f, pl.ds(HIST - LANES, LANES)]
            gid2[f, pl.ds(0, LANES)] = xa
            gid2[f, pl.ds(LANES, LANES)] = xb
            gid2[f, pl.ds(2 * LANES, LANES)] = xc
            gid2[f, pl.ds(HIST - LANES, LANES)] = xd
            cnt = (plsc.all_reduce_population_count(xa != 0)
                   + plsc.all_reduce_population_count(xb != 0)
                   + plsc.all_reduce_population_count(xc != 0)
                   + plsc.all_reduce_population_count((xd != 0) & tail2))
            rcp[f, pl.ds(0, LANES)] = 1.0 / jnp.maximum(
                cnt.astype(jnp.float32), 1.0)

    def fire(gid2, rows, sem):
        for f in range(NUM_FIELDS):
            pltpu.async_copy(
                tab_hbm.at[f].at[gid2.at[f]], rows.at[f], sem)

    def drain_reduce(b, gid2, rows, rcp, sem):
        for f in range(NUM_FIELDS):
            pltpu.make_async_copy(
                tab_hbm.at[f].at[gid2.at[f]], rows.at[f], sem).wait()
        for f in range(NUM_FIELDS):
            rf = rcp[f, pl.ds(0, LANES)]

            @pl.loop(0, HIST, init_carry=(zero16, zero16), unroll=5)
            def _sum(l, carry):
                a0, a1 = carry
                a0 = a0 + rows[f, l, pl.ds(0, HALF)]
                a1 = a1 + rows[f, l, pl.ds(HALF, HALF)]
                return a0, a1

            a0, a1 = _sum
            outv[pl.ds(f * DIM, HALF)] = a0 * rf
            outv[pl.ds(f * DIM + HALF, HALF)] = a1 * rf
        pltpu.sync_copy(outv, out_hbm.at[base + b])

    def stage_full(b, xv, gid2, rows, rcp, sem):
        stage(b, xv, gid2, rcp, sem)
        fire(gid2, rows, sem)

    for g2 in (gid0, gid1):
        for f in range(NUM_FIELDS):
            g2[f, pl.ds(HIST_PAD - LANES, LANES)] = jnp.zeros(
                (LANES,), jnp.int32)

    stage_full(0, xv0, gid0, rows0, rcp0, sem0)

    @pl.loop(0, B_PER_W // 2 - 1)
    def _pair(t):
        b = 2 * t
        stage_full(b + 1, xv1, gid1, rows1, rcp1, sem1)
        drain_reduce(b, gid0, rows0, rcp0, sem0)
        stage_full(b + 2, xv0, gid0, rows0, rcp0, sem0)
        drain_reduce(b + 1, gid1, rows1, rcp1, sem1)

    stage_full(B_PER_W - 1, xv1, gid1, rows1, rcp1, sem1)
    drain_reduce(B_PER_W - 2, gid0, rows0, rcp0, sem0)
    drain_reduce(B_PER_W - 1, gid1, rows1, rcp1, sem1)


@jax.jit
def kernel(x, tables):
    call = pl.kernel(
        _body,
        out_type=jax.ShapeDtypeStruct((BATCH, OUT_D), jnp.float32),
        mesh=plsc.VectorSubcoreMesh(core_axis_name="c", subcore_axis_name="s"),
        compiler_params=pltpu.CompilerParams(
            needs_layout_passes=False, use_tc_tiling_on_sc=False),
        scratch_types=[
            pltpu.VMEM((NUM_FIELDS, HIST), jnp.int32),
            pltpu.VMEM((NUM_FIELDS, HIST), jnp.int32),
            pltpu.VMEM((NUM_FIELDS, HIST_PAD), jnp.int32),
            pltpu.VMEM((NUM_FIELDS, HIST_PAD), jnp.int32),
            pltpu.VMEM((NUM_FIELDS, HIST_PAD, DIM), jnp.float32),
            pltpu.VMEM((NUM_FIELDS, HIST_PAD, DIM), jnp.float32),
            pltpu.VMEM((NUM_FIELDS, LANES), jnp.float32),
            pltpu.VMEM((NUM_FIELDS, LANES), jnp.float32),
            pltpu.VMEM((OUT_D,), jnp.float32),
            pltpu.SemaphoreType.DMA,
            pltpu.SemaphoreType.DMA,
        ],
    )
    return call(x, tables)
